# Initial kernel scaffold; baseline (speedup 1.0000x reference)
import jax
import jax.numpy as jnp
from jax.experimental import pallas as pl


def kernel(atom_features, bond_features, pair_indices, kernel, bias):
    raise NotImplementedError("write your pallas kernel here")


# speedup vs baseline: 3.6939x; 3.6939x over previous
import functools

import jax
import jax.numpy as jnp
from jax import lax
from jax.experimental import pallas as pl
from jax.experimental.pallas import tpu as pltpu
from jax.experimental.pallas import tpu_sc as plsc

N_NODES = 50000
ATOM_DIM = 32
BOND_DIM = 16

NC = 2
NS = 16
NW = NC * NS

E_PAD = 100352
GATHER_CHUNK = E_PAD // NW
TC_BLOCK = 2048
TC_GRID = E_PAD // TC_BLOCK
SCAT_CHUNK = E_PAD // NS
SCAT_SUB = SCAT_CHUNK // 2
ACC_ROWS = N_NODES + 48
ZERO_ROWS = ACC_ROWS // NS
OUT_ROWS = N_NODES // NS
HALF = ATOM_DIM // 2


def _gather_body(atom_hbm, idx_hbm, out_hbm, idx_v, rows_v, sem):
    wid = lax.axis_index("s") * NC + lax.axis_index("c")
    base = wid * GATHER_CHUNK
    pltpu.sync_copy(idx_hbm.at[pl.ds(base, GATHER_CHUNK)], idx_v)
    pltpu.async_copy(atom_hbm.at[idx_v], rows_v, sem).wait()
    pltpu.sync_copy(rows_v, out_hbm.at[pl.ds(base, GATHER_CHUNK)])


def _sc_gather(atom_features, nbr_idx_padded):
    mesh = plsc.VectorSubcoreMesh(core_axis_name="c", subcore_axis_name="s")
    k = functools.partial(
        pl.kernel,
        mesh=mesh,
        out_type=jax.ShapeDtypeStruct((E_PAD, ATOM_DIM), jnp.float32),
        scratch_types=[
            pltpu.VMEM((GATHER_CHUNK,), jnp.int32),
            pltpu.VMEM((GATHER_CHUNK, ATOM_DIM), jnp.float32),
            pltpu.SemaphoreType.DMA,
        ],
        compiler_params=pltpu.CompilerParams(use_tc_tiling_on_sc=False),
    )(_gather_body)
    return k(atom_features, nbr_idx_padded)


def _tc_body(bond_ref, nbr_ref, wt2_ref, r_ref, f_ref, b2t_ref,
             tlo_ref, thi_ref):
    bond = bond_ref[...]
    nbr = nbr_ref[...]
    bond_rep = jnp.dot(bond, r_ref[...], preferred_element_type=jnp.float32)
    g = jnp.dot(nbr, wt2_ref[...], preferred_element_type=jnp.float32)
    tr = jnp.dot(bond_rep * g, f_ref[...], preferred_element_type=jnp.float32)
    tr = tr + jnp.dot(nbr, b2t_ref[...], preferred_element_type=jnp.float32)
    tlo_ref[...] = tr[:, :HALF]
    thi_ref[...] = tr[:, HALF:]


def _tc_transform(bond_padded, nbr_feats, wt2, r, f, b2t):
    out_shape = [
        jax.ShapeDtypeStruct((E_PAD, HALF), jnp.float32),
        jax.ShapeDtypeStruct((E_PAD, HALF), jnp.float32),
    ]
    kdim = BOND_DIM * ATOM_DIM
    return pl.pallas_call(
        _tc_body,
        grid=(TC_GRID,),
        in_specs=[
            pl.BlockSpec((TC_BLOCK, BOND_DIM), lambda i: (i, 0)),
            pl.BlockSpec((TC_BLOCK, ATOM_DIM), lambda i: (i, 0)),
            pl.BlockSpec((ATOM_DIM, kdim), lambda i: (0, 0)),
            pl.BlockSpec((BOND_DIM, kdim), lambda i: (0, 0)),
            pl.BlockSpec((kdim, ATOM_DIM), lambda i: (0, 0)),
            pl.BlockSpec((ATOM_DIM, ATOM_DIM), lambda i: (0, 0)),
        ],
        out_specs=[
            pl.BlockSpec((TC_BLOCK, HALF), lambda i: (i, 0)),
            pl.BlockSpec((TC_BLOCK, HALF), lambda i: (i, 0)),
        ],
        out_shape=out_shape,
    )(bond_padded, nbr_feats, wt2, r, f, b2t)


def _scatter_body(src_hbm, tlo_hbm, thi_hbm, zeros_hbm, outlo_hbm, outhi_hbm,
                  acc, idx_v, rows_v):
    cid = lax.axis_index("c")
    sid = lax.axis_index("s")
    pltpu.sync_copy(zeros_hbm, acc.at[pl.ds(sid * ZERO_ROWS, ZERO_ROWS)])
    plsc.subcore_barrier()
    for sub in range(SCAT_CHUNK // SCAT_SUB):
        base = sid * SCAT_CHUNK + sub * SCAT_SUB
        pltpu.sync_copy(src_hbm.at[pl.ds(base, SCAT_SUB)], idx_v)

        @pl.when(cid == 0)
        def _():
            pltpu.sync_copy(tlo_hbm.at[pl.ds(base, SCAT_SUB)], rows_v)

        @pl.when(cid == 1)
        def _():
            pltpu.sync_copy(thi_hbm.at[pl.ds(base, SCAT_SUB)], rows_v)

        pltpu.sync_copy(rows_v, acc.at[idx_v], add=True)
    plsc.subcore_barrier()
    obase = sid * OUT_ROWS

    @pl.when(cid == 0)
    def _():
        pltpu.sync_copy(acc.at[pl.ds(obase, OUT_ROWS)],
                        outlo_hbm.at[pl.ds(obase, OUT_ROWS)])

    @pl.when(cid == 1)
    def _():
        pltpu.sync_copy(acc.at[pl.ds(obase, OUT_ROWS)],
                        outhi_hbm.at[pl.ds(obase, OUT_ROWS)])


def _sc_scatter(src_padded, tlo, thi, zeros_block):
    mesh = plsc.VectorSubcoreMesh(core_axis_name="c", subcore_axis_name="s")
    k = functools.partial(
        pl.kernel,
        mesh=mesh,
        out_type=(
            jax.ShapeDtypeStruct((N_NODES, HALF), jnp.float32),
            jax.ShapeDtypeStruct((N_NODES, HALF), jnp.float32),
        ),
        scratch_types=[
            pltpu.VMEM_SHARED((ACC_ROWS, HALF), jnp.float32),
            pltpu.VMEM((SCAT_SUB,), jnp.int32),
            pltpu.VMEM((SCAT_SUB, HALF), jnp.float32),
        ],
        compiler_params=pltpu.CompilerParams(use_tc_tiling_on_sc=False),
    )(_scatter_body)
    return k(src_padded, tlo, thi, zeros_block)


def kernel(atom_features, bond_features, pair_indices, kernel, bias):
    e = pair_indices.shape[0]
    pad = E_PAD - e
    src = jnp.pad(pair_indices[:, 0], (0, pad), constant_values=N_NODES)
    nbr = jnp.pad(pair_indices[:, 1], (0, pad), constant_values=0)
    bond_p = jnp.pad(bond_features, ((0, pad), (0, 0)))
    kdim = BOND_DIM * ATOM_DIM
    wt2 = kernel.reshape(BOND_DIM, ATOM_DIM, ATOM_DIM).transpose(2, 0, 1)
    wt2 = wt2.reshape(ATOM_DIM, kdim)
    b2t = bias.reshape(ATOM_DIM, ATOM_DIM).T
    c_ids = jnp.arange(kdim, dtype=jnp.int32)
    r = (c_ids[None, :] // ATOM_DIM
         == jnp.arange(BOND_DIM, dtype=jnp.int32)[:, None]).astype(jnp.float32)
    f = (c_ids[:, None] % ATOM_DIM
         == jnp.arange(ATOM_DIM, dtype=jnp.int32)[None, :]).astype(jnp.float32)
    zeros_block = jnp.zeros((ZERO_ROWS, HALF), jnp.float32)

    nbr_feats = _sc_gather(atom_features, nbr)
    tlo, thi = _tc_transform(bond_p, nbr_feats, wt2, r, f, b2t)
    out_lo, out_hi = _sc_scatter(src, tlo, thi, zeros_block)
    return jnp.concatenate([out_lo, out_hi], axis=1)


# speedup vs baseline: 4.0550x; 1.0977x over previous
import functools

import jax
import jax.numpy as jnp
from jax import lax
from jax.experimental import pallas as pl
from jax.experimental.pallas import tpu as pltpu
from jax.experimental.pallas import tpu_sc as plsc

N_NODES = 50000
ATOM_DIM = 32
BOND_DIM = 16
N_EDGES = 100000

NC = 2
NS = 16
NW = NC * NS

G_CHUNK = 3128
G_TAIL = N_EDGES - (NW - 1) * G_CHUNK

TC_BLOCK = 2048
TC_GRID = (N_EDGES + TC_BLOCK - 1) // TC_BLOCK

S_CHUNK = 3136
S_NCHUNK = 32
S_TAIL = N_EDGES - (S_NCHUNK - 1) * S_CHUNK
ACC_ROWS = 50048
ZERO_ROWS = ACC_ROWS // NS
OUT_ROWS = N_NODES // NS
HALF = ATOM_DIM // 2


def _gather_body(atom_hbm, idx_hbm, out_hbm, idx_v, rows_v, sem):
    wid = lax.axis_index("s") * NC + lax.axis_index("c")
    base = wid * G_CHUNK

    @pl.when(wid < NW - 1)
    def _():
        pltpu.sync_copy(idx_hbm.at[pl.ds(base, G_CHUNK)],
                        idx_v.at[pl.ds(0, G_CHUNK)])
        pltpu.async_copy(atom_hbm.at[idx_v.at[pl.ds(0, G_CHUNK)]],
                         rows_v.at[pl.ds(0, G_CHUNK)], sem).wait()
        pltpu.sync_copy(rows_v.at[pl.ds(0, G_CHUNK)],
                        out_hbm.at[pl.ds(base, G_CHUNK)])

    @pl.when(wid == NW - 1)
    def _():
        pltpu.sync_copy(idx_hbm.at[pl.ds(base, G_TAIL)],
                        idx_v.at[pl.ds(0, G_TAIL)])
        pltpu.async_copy(atom_hbm.at[idx_v.at[pl.ds(0, G_TAIL)]],
                         rows_v.at[pl.ds(0, G_TAIL)], sem).wait()
        pltpu.sync_copy(rows_v.at[pl.ds(0, G_TAIL)],
                        out_hbm.at[pl.ds(base, G_TAIL)])


def _sc_gather(atom_features, nbr_idx):
    mesh = plsc.VectorSubcoreMesh(core_axis_name="c", subcore_axis_name="s")
    k = functools.partial(
        pl.kernel,
        mesh=mesh,
        out_type=jax.ShapeDtypeStruct((N_EDGES, ATOM_DIM), jnp.float32),
        scratch_types=[
            pltpu.VMEM((G_CHUNK,), jnp.int32),
            pltpu.VMEM((G_CHUNK, ATOM_DIM), jnp.float32),
            pltpu.SemaphoreType.DMA,
        ],
        compiler_params=pltpu.CompilerParams(use_tc_tiling_on_sc=False),
    )(_gather_body)
    return k(atom_features, nbr_idx)


def _tc_body(bond_ref, nbr_ref, wt2_ref, r_ref, f_ref, b2t_ref,
             tlo_ref, thi_ref):
    bond = bond_ref[...]
    nbr = nbr_ref[...]
    bond_rep = jnp.dot(bond, r_ref[...], preferred_element_type=jnp.float32)
    g = jnp.dot(nbr, wt2_ref[...], preferred_element_type=jnp.float32)
    tr = jnp.dot(bond_rep * g, f_ref[...], preferred_element_type=jnp.float32)
    tr = tr + jnp.dot(nbr, b2t_ref[...], preferred_element_type=jnp.float32)
    tlo_ref[...] = tr[:, :HALF]
    thi_ref[...] = tr[:, HALF:]


def _tc_transform(bond_features, nbr_feats, wt2, r, f, b2t):
    out_shape = [
        jax.ShapeDtypeStruct((N_EDGES, HALF), jnp.float32),
        jax.ShapeDtypeStruct((N_EDGES, HALF), jnp.float32),
    ]
    kdim = BOND_DIM * ATOM_DIM
    return pl.pallas_call(
        _tc_body,
        grid=(TC_GRID,),
        in_specs=[
            pl.BlockSpec((TC_BLOCK, BOND_DIM), lambda i: (i, 0)),
            pl.BlockSpec((TC_BLOCK, ATOM_DIM), lambda i: (i, 0)),
            pl.BlockSpec((ATOM_DIM, kdim), lambda i: (0, 0)),
            pl.BlockSpec((BOND_DIM, kdim), lambda i: (0, 0)),
            pl.BlockSpec((kdim, ATOM_DIM), lambda i: (0, 0)),
            pl.BlockSpec((ATOM_DIM, ATOM_DIM), lambda i: (0, 0)),
        ],
        out_specs=[
            pl.BlockSpec((TC_BLOCK, HALF), lambda i: (i, 0)),
            pl.BlockSpec((TC_BLOCK, HALF), lambda i: (i, 0)),
        ],
        out_shape=out_shape,
    )(bond_features, nbr_feats, wt2, r, f, b2t)


def _scatter_chunk(src_hbm, t_hbm, acc, idx_v, rows_v, base, size):
    pltpu.sync_copy(src_hbm.at[pl.ds(base, size)], idx_v.at[pl.ds(0, size)])
    pltpu.sync_copy(t_hbm.at[pl.ds(base, size)], rows_v.at[pl.ds(0, size)])
    pltpu.sync_copy(rows_v.at[pl.ds(0, size)],
                    acc.at[idx_v.at[pl.ds(0, size)]], add=True)


def _scatter_body(src_hbm, tlo_hbm, thi_hbm, zeros_hbm, out_hbm,
                  acc, idx_v, rows_v):
    cid = lax.axis_index("c")
    sid = lax.axis_index("s")
    pltpu.sync_copy(zeros_hbm, acc.at[pl.ds(sid * ZERO_ROWS, ZERO_ROWS)])
    plsc.subcore_barrier()

    def do(base, size):
        @pl.when(cid == 0)
        def _():
            _scatter_chunk(src_hbm, tlo_hbm, acc, idx_v, rows_v, base, size)

        @pl.when(cid == 1)
        def _():
            _scatter_chunk(src_hbm, thi_hbm, acc, idx_v, rows_v, base, size)

    do(sid * S_CHUNK, S_CHUNK)

    @pl.when(sid < NS - 1)
    def _():
        do((NS + sid) * S_CHUNK, S_CHUNK)

    @pl.when(sid == NS - 1)
    def _():
        do((S_NCHUNK - 1) * S_CHUNK, S_TAIL)

    plsc.subcore_barrier()
    obase = sid * OUT_ROWS
    pltpu.sync_copy(acc.at[pl.ds(obase, OUT_ROWS)],
                    out_hbm.at[pl.ds(obase, OUT_ROWS), pl.ds(cid * HALF, HALF)])


def _sc_scatter(src, tlo, thi, zeros_block):
    mesh = plsc.VectorSubcoreMesh(core_axis_name="c", subcore_axis_name="s")
    k = functools.partial(
        pl.kernel,
        mesh=mesh,
        out_type=jax.ShapeDtypeStruct((N_NODES, ATOM_DIM), jnp.float32),
        scratch_types=[
            pltpu.VMEM_SHARED((ACC_ROWS, HALF), jnp.float32),
            pltpu.VMEM((S_CHUNK,), jnp.int32),
            pltpu.VMEM((S_CHUNK, HALF), jnp.float32),
        ],
        compiler_params=pltpu.CompilerParams(use_tc_tiling_on_sc=False),
    )(_scatter_body)
    return k(src, tlo, thi, zeros_block)


def kernel(atom_features, bond_features, pair_indices, kernel, bias):
    src = pair_indices[:, 0]
    nbr = pair_indices[:, 1]
    kdim = BOND_DIM * ATOM_DIM
    wt2 = kernel.reshape(BOND_DIM, ATOM_DIM, ATOM_DIM).transpose(2, 0, 1)
    wt2 = wt2.reshape(ATOM_DIM, kdim)
    b2t = bias.reshape(ATOM_DIM, ATOM_DIM).T
    c_ids = jnp.arange(kdim, dtype=jnp.int32)
    r = (c_ids[None, :] // ATOM_DIM
         == jnp.arange(BOND_DIM, dtype=jnp.int32)[:, None]).astype(jnp.float32)
    f = (c_ids[:, None] % ATOM_DIM
         == jnp.arange(ATOM_DIM, dtype=jnp.int32)[None, :]).astype(jnp.float32)
    zeros_block = jnp.zeros((ZERO_ROWS, HALF), jnp.float32)

    nbr_feats = _sc_gather(atom_features, nbr)
    tlo, thi = _tc_transform(bond_features, nbr_feats, wt2, r, f, b2t)
    return _sc_scatter(src, tlo, thi, zeros_block)
